# Initial kernel scaffold; baseline (speedup 1.0000x reference)
import jax
import jax.numpy as jnp
from jax.experimental import pallas as pl


def kernel(x, in_w, conv_w, conv_b, x_w, dt_w, dt_b, A_log, D, out_w, norm_w, norm_f_w):
    raise NotImplementedError("write your pallas kernel here")


# speedup vs baseline: 9.3824x; 9.3824x over previous
import functools

import jax
import jax.numpy as jnp
from jax.experimental import pallas as pl
from jax.experimental.pallas import tpu as pltpu

D_MODEL = 1024
N_LAYERS = 4
D_CONV = 4
D_STATE = 16
INTER = 2048
DT_RANK = 64
B_SZ = 4
SEQ = 1024
EPS = 1e-5

TQ = 256
SQ = 128


def _rms(v, w):
    return v * jax.lax.rsqrt(jnp.mean(v * v, axis=-1, keepdims=True) + EPS) * w


def _sigmoid(v):
    return 1.0 / (1.0 + jnp.exp(-v))


def _ka_body(h_ref, hp_ref, nw_ref, inw_ref, cw_ref, cb_ref, xw_ref, dtw_ref,
             dtb_ref, dvec_ref, dt_o, xt_o, sg_o, z1_o, bct_o):
    c = pl.program_id(1)
    nw = nw_ref[...]
    xb = h_ref[0]
    nx = _rms(xb, nw)
    proj = jnp.dot(nx.astype(jnp.bfloat16), inw_ref[...],
                   preferred_element_type=jnp.float32)
    hs = proj[:, :INTER]
    gate = proj[:, INTER:]

    hp = hp_ref[0, TQ - 8:, :]
    hs_h = jnp.dot(_rms(hp, nw).astype(jnp.bfloat16), inw_ref[:, :INTER],
                   preferred_element_type=jnp.float32)
    hs_h = hs_h * jnp.where(c > 0, 1.0, 0.0)

    full = jnp.concatenate([hs_h, hs], axis=0)
    acc = full[8:8 + TQ] * cw_ref[3:4] + cb_ref[...]
    acc = acc + full[5:5 + TQ] * cw_ref[0:1]
    acc = acc + full[6:6 + TQ] * cw_ref[1:2]
    acc = acc + full[7:7 + TQ] * cw_ref[2:3]
    u = acc * _sigmoid(acc)

    ssm = jnp.dot(u.astype(jnp.bfloat16), xw_ref[...],
                  preferred_element_type=jnp.float32)
    dt_in = ssm[:, :DT_RANK]
    bc = ssm[:, DT_RANK:DT_RANK + 2 * D_STATE]
    dtz = jnp.dot(dt_in, dtw_ref[...],
                  preferred_element_type=jnp.float32) + dtb_ref[...]
    dtp = jnp.maximum(dtz, 0.0) + jnp.log1p(jnp.exp(-jnp.abs(dtz)))
    sgv = gate * _sigmoid(gate)

    dt_o[0] = dtp
    xt_o[0] = dtp * u
    sg_o[0] = sgv
    z1_o[0] = u * dvec_ref[...] * sgv

    ii = jax.lax.broadcasted_iota(jnp.int32, (TQ, TQ), 0)
    jj = jax.lax.broadcasted_iota(jnp.int32, (TQ, TQ), 1)
    eye = jnp.where(ii == jj, 1.0, 0.0)
    bct_o[0] = jax.lax.dot_general(bc, eye, (((0,), (0,)), ((), ())),
                                   preferred_element_type=jnp.float32)


def _layer_parallel(h, nw, inwT, cwT, cb, xwT, dtwT, dtb, dvec, interpret=False):
    grid = (B_SZ, SEQ // TQ)
    oi = jax.ShapeDtypeStruct((B_SZ, SEQ, INTER), jnp.float32)
    return pl.pallas_call(
        _ka_body,
        grid=grid,
        in_specs=[
            pl.BlockSpec((1, TQ, D_MODEL), lambda b, c: (b, c, 0)),
            pl.BlockSpec((1, TQ, D_MODEL), lambda b, c: (b, jnp.maximum(c - 1, 0), 0)),
            pl.BlockSpec((1, D_MODEL), lambda b, c: (0, 0)),
            pl.BlockSpec((D_MODEL, 2 * INTER), lambda b, c: (0, 0)),
            pl.BlockSpec((D_CONV, INTER), lambda b, c: (0, 0)),
            pl.BlockSpec((1, INTER), lambda b, c: (0, 0)),
            pl.BlockSpec((INTER, 128), lambda b, c: (0, 0)),
            pl.BlockSpec((DT_RANK, INTER), lambda b, c: (0, 0)),
            pl.BlockSpec((1, INTER), lambda b, c: (0, 0)),
            pl.BlockSpec((1, INTER), lambda b, c: (0, 0)),
        ],
        out_specs=[
            pl.BlockSpec((1, TQ, INTER), lambda b, c: (b, c, 0)),
            pl.BlockSpec((1, TQ, INTER), lambda b, c: (b, c, 0)),
            pl.BlockSpec((1, TQ, INTER), lambda b, c: (b, c, 0)),
            pl.BlockSpec((1, TQ, INTER), lambda b, c: (b, c, 0)),
            pl.BlockSpec((1, 2 * D_STATE, TQ), lambda b, c: (b, 0, c)),
        ],
        out_shape=[oi, oi, oi, oi,
                   jax.ShapeDtypeStruct((B_SZ, 2 * D_STATE, SEQ), jnp.float32)],
        compiler_params=pltpu.CompilerParams(
            dimension_semantics=("parallel", "arbitrary"),
            vmem_limit_bytes=56 * 1024 * 1024,
        ),
        name="mamba_tok",
        interpret=interpret,
    )(h, h, nw, inwT, cwT, cb, xwT, dtwT, dtb, dvec)


def _kb_body(dt_ref, xt_ref, sg_ref, z1_ref, bct_ref, at_ref, ow_ref, hr_ref,
             nfw_ref, ho_ref, y_scr, st_scr, *, final):
    c = pl.program_id(1)

    @pl.when(c == 0)
    def _():
        st_scr[...] = jnp.zeros_like(st_scr)

    st0 = st_scr[...]
    bcv = bct_ref[0]
    atv = at_ref[...]

    def group(to, st):
        base = pl.multiple_of(to * 8, 8)
        gd = dt_ref[0, pl.ds(base, 8), :]
        gx = xt_ref[0, pl.ds(base, 8), :]
        rolled = pltpu.roll(bcv, SQ - base, axis=1)
        ys = []
        for r in range(8):
            drow = gd[r:r + 1]
            xrow = gx[r:r + 1]
            col = rolled[:, r:r + 1]
            bcol = col[0:D_STATE]
            ccol = col[D_STATE:2 * D_STATE]
            da = jnp.exp(drow * atv)
            st = da * st + xrow * bcol
            ys.append(jnp.sum(st * ccol, axis=0, keepdims=True))
        y_scr[pl.ds(base, 8), :] = jnp.concatenate(ys, axis=0)
        return st

    stf = jax.lax.fori_loop(0, SQ // 8, group, st0)
    st_scr[...] = stf

    post = (y_scr[...] * sg_ref[0] + z1_ref[0]).astype(jnp.bfloat16)
    res = jnp.dot(post, ow_ref[...], preferred_element_type=jnp.float32) + hr_ref[0]
    if final:
        res = _rms(res, nfw_ref[...])
    ho_ref[0] = res


def _layer_scan(h, dt, xt, sg, z1, bct, atT, owT, nfw, final, interpret=False):
    grid = (B_SZ, SEQ // SQ)
    return pl.pallas_call(
        functools.partial(_kb_body, final=final),
        grid=grid,
        in_specs=[
            pl.BlockSpec((1, SQ, INTER), lambda b, c: (b, c, 0)),
            pl.BlockSpec((1, SQ, INTER), lambda b, c: (b, c, 0)),
            pl.BlockSpec((1, SQ, INTER), lambda b, c: (b, c, 0)),
            pl.BlockSpec((1, SQ, INTER), lambda b, c: (b, c, 0)),
            pl.BlockSpec((1, 2 * D_STATE, SQ), lambda b, c: (b, 0, c)),
            pl.BlockSpec((D_STATE, INTER), lambda b, c: (0, 0)),
            pl.BlockSpec((INTER, D_MODEL), lambda b, c: (0, 0)),
            pl.BlockSpec((1, SQ, D_MODEL), lambda b, c: (b, c, 0)),
            pl.BlockSpec((1, D_MODEL), lambda b, c: (0, 0)),
        ],
        out_specs=pl.BlockSpec((1, SQ, D_MODEL), lambda b, c: (b, c, 0)),
        out_shape=jax.ShapeDtypeStruct((B_SZ, SEQ, D_MODEL), jnp.float32),
        scratch_shapes=[
            pltpu.VMEM((SQ, INTER), jnp.float32),
            pltpu.VMEM((D_STATE, INTER), jnp.float32),
        ],
        compiler_params=pltpu.CompilerParams(
            dimension_semantics=("parallel", "arbitrary"),
            vmem_limit_bytes=48 * 1024 * 1024,
        ),
        name="mamba_scan",
        interpret=interpret,
    )(dt, xt, sg, z1, bct, atT, owT, h, nfw)


def kernel(x, in_w, conv_w, conv_b, x_w, dt_w, dt_b, A_log, D, out_w, norm_w,
           norm_f_w, interpret=False):
    f32 = jnp.float32
    h = x
    nfw = norm_f_w.reshape(1, D_MODEL)
    for l in range(N_LAYERS):
        inwT = in_w[l].T.astype(jnp.bfloat16)
        cwT = conv_w[l].T
        cb = conv_b[l].reshape(1, INTER)
        xwT = jnp.pad(x_w[l].T, ((0, 0), (0, 128 - (DT_RANK + 2 * D_STATE)))
                      ).astype(jnp.bfloat16)
        dtwT = dt_w[l].T.astype(f32)
        dtb = dt_b[l].reshape(1, INTER)
        dvec = D[l].reshape(1, INTER)
        nw = norm_w[l].reshape(1, D_MODEL)
        atT = (-jnp.exp(A_log[l])).T.astype(f32)
        owT = out_w[l].T.astype(jnp.bfloat16)

        dt_a, xt_a, sg_a, z1_a, bct = _layer_parallel(
            h, nw, inwT, cwT, cb, xwT, dtwT, dtb, dvec, interpret=interpret)
        h = _layer_scan(h, dt_a, xt_a, sg_a, z1_a, bct, atT, owT, nfw,
                        final=(l == N_LAYERS - 1), interpret=interpret)
    return h


# speedup vs baseline: 10.6681x; 1.1370x over previous
import functools

import jax
import jax.numpy as jnp
from jax.experimental import pallas as pl
from jax.experimental.pallas import tpu as pltpu

D_MODEL = 1024
N_LAYERS = 4
D_CONV = 4
D_STATE = 16
INTER = 2048
DT_RANK = 64
B_SZ = 4
SEQ = 1024
EPS = 1e-5

TQ = 256
SQ = 256


def _rms(v, w):
    return v * jax.lax.rsqrt(jnp.mean(v * v, axis=-1, keepdims=True) + EPS) * w


def _sigmoid(v):
    return 1.0 / (1.0 + jnp.exp(-v))


def _ka_body(h_ref, hp_ref, nw_ref, inw_ref, cw_ref, cb_ref, xw_ref, dtw_ref,
             dtb_ref, dvec_ref, dt_o, xt_o, sg_o, z1_o, bct_o):
    c = pl.program_id(1)
    nw = nw_ref[...]
    xb = jnp.concatenate([hp_ref[0, TQ - 8:, :], h_ref[0]], axis=0)
    nx = _rms(xb, nw)
    proj = jnp.dot(nx.astype(jnp.bfloat16), inw_ref[...],
                   preferred_element_type=jnp.float32)
    hs = proj[8:, :INTER]
    gate = proj[8:, INTER:]
    hs_h = proj[0:8, :INTER] * jnp.where(c > 0, 1.0, 0.0)

    full = jnp.concatenate([hs_h, hs], axis=0)
    acc = full[8:8 + TQ] * cw_ref[3:4] + cb_ref[...]
    acc = acc + full[5:5 + TQ] * cw_ref[0:1]
    acc = acc + full[6:6 + TQ] * cw_ref[1:2]
    acc = acc + full[7:7 + TQ] * cw_ref[2:3]
    u = acc * _sigmoid(acc)

    ssm = jnp.dot(u.astype(jnp.bfloat16), xw_ref[...],
                  preferred_element_type=jnp.float32)
    dt_in = ssm[:, :DT_RANK]
    bc = ssm[:, DT_RANK:DT_RANK + 2 * D_STATE]
    dtz = jnp.dot(dt_in, dtw_ref[...],
                  preferred_element_type=jnp.float32) + dtb_ref[...]
    dtp = jnp.maximum(dtz, 0.0) + jnp.log1p(jnp.exp(-jnp.abs(dtz)))
    sgv = gate * _sigmoid(gate)

    dt_o[0] = dtp.astype(jnp.bfloat16)
    xt_o[0] = (dtp * u).astype(jnp.bfloat16)
    sg_o[0] = sgv.astype(jnp.bfloat16)
    z1_o[0] = (u * dvec_ref[...] * sgv).astype(jnp.bfloat16)

    ii = jax.lax.broadcasted_iota(jnp.int32, (TQ, TQ), 0)
    jj = jax.lax.broadcasted_iota(jnp.int32, (TQ, TQ), 1)
    eye = jnp.where(ii == jj, 1.0, 0.0)
    bct_o[0] = jax.lax.dot_general(bc, eye, (((0,), (0,)), ((), ())),
                                   preferred_element_type=jnp.float32)


def _layer_parallel(h, nw, inwT, cwT, cb, xwT, dtwT, dtb, dvec, interpret=False):
    grid = (B_SZ, SEQ // TQ)
    oi = jax.ShapeDtypeStruct((B_SZ, SEQ, INTER), jnp.bfloat16)
    return pl.pallas_call(
        _ka_body,
        grid=grid,
        in_specs=[
            pl.BlockSpec((1, TQ, D_MODEL), lambda b, c: (b, c, 0)),
            pl.BlockSpec((1, TQ, D_MODEL), lambda b, c: (b, jnp.maximum(c - 1, 0), 0)),
            pl.BlockSpec((1, D_MODEL), lambda b, c: (0, 0)),
            pl.BlockSpec((D_MODEL, 2 * INTER), lambda b, c: (0, 0)),
            pl.BlockSpec((D_CONV, INTER), lambda b, c: (0, 0)),
            pl.BlockSpec((1, INTER), lambda b, c: (0, 0)),
            pl.BlockSpec((INTER, 128), lambda b, c: (0, 0)),
            pl.BlockSpec((DT_RANK, INTER), lambda b, c: (0, 0)),
            pl.BlockSpec((1, INTER), lambda b, c: (0, 0)),
            pl.BlockSpec((1, INTER), lambda b, c: (0, 0)),
        ],
        out_specs=[
            pl.BlockSpec((1, TQ, INTER), lambda b, c: (b, c, 0)),
            pl.BlockSpec((1, TQ, INTER), lambda b, c: (b, c, 0)),
            pl.BlockSpec((1, TQ, INTER), lambda b, c: (b, c, 0)),
            pl.BlockSpec((1, TQ, INTER), lambda b, c: (b, c, 0)),
            pl.BlockSpec((1, 2 * D_STATE, TQ), lambda b, c: (b, 0, c)),
        ],
        out_shape=[oi, oi, oi, oi,
                   jax.ShapeDtypeStruct((B_SZ, 2 * D_STATE, SEQ), jnp.float32)],
        compiler_params=pltpu.CompilerParams(
            dimension_semantics=("parallel", "arbitrary"),
            vmem_limit_bytes=56 * 1024 * 1024,
        ),
        name="mamba_tok",
        interpret=interpret,
    )(h, h, nw, inwT, cwT, cb, xwT, dtwT, dtb, dvec)


def _kb_body(dt_ref, xt_ref, sg_ref, z1_ref, bct_ref, at_ref, ow_ref, hr_ref,
             nfw_ref, ho_ref, y_scr, st_scr, *, final):
    c = pl.program_id(1)

    @pl.when(c == 0)
    def _():
        st_scr[...] = jnp.zeros_like(st_scr)

    bcv = bct_ref[0]

    NQ = 2
    IW = INTER // NQ
    for q in range(NQ):
        lo = q * IW
        atq = at_ref[:, lo:lo + IW]

        def group(to, carry, lo=lo, atq=atq):
            st, rolled = carry
            base = pl.multiple_of(to * 8, 8)
            gd = dt_ref[0, pl.ds(base, 8), lo:lo + IW].astype(jnp.float32)
            gx = xt_ref[0, pl.ds(base, 8), lo:lo + IW]
            rolled_nxt = pltpu.roll(bcv, SQ - 8 - base, axis=1)
            ys = []
            for r in range(8):
                drow = gd[r:r + 1]
                xrow = gx[r:r + 1]
                col = rolled[:, r:r + 1]
                bcol = col[0:D_STATE].astype(jnp.bfloat16)
                ccol = col[D_STATE:2 * D_STATE].astype(jnp.bfloat16)
                da = jnp.exp2(drow * atq).astype(jnp.bfloat16)
                st = da * st + xrow * bcol
                ys.append(jnp.sum(st * ccol, axis=0, keepdims=True))
            y_scr[pl.ds(base, 8), lo:lo + IW] = (
                jnp.concatenate(ys, axis=0).astype(jnp.float32))
            return (st, rolled_nxt)

        stqf, _ = jax.lax.fori_loop(
            0, SQ // 8, group,
            (st_scr[:, lo:lo + IW].astype(jnp.bfloat16), bcv))
        st_scr[:, lo:lo + IW] = stqf.astype(jnp.float32)

    post = y_scr[...].astype(jnp.bfloat16) * sg_ref[0] + z1_ref[0]
    res = jnp.dot(post, ow_ref[...], preferred_element_type=jnp.float32) + hr_ref[0]
    if final:
        res = _rms(res, nfw_ref[...])
    ho_ref[0] = res


def _layer_scan(h, dt, xt, sg, z1, bct, atT, owT, nfw, final, interpret=False):
    grid = (B_SZ, SEQ // SQ)
    return pl.pallas_call(
        functools.partial(_kb_body, final=final),
        grid=grid,
        in_specs=[
            pl.BlockSpec((1, SQ, INTER), lambda b, c: (b, c, 0)),
            pl.BlockSpec((1, SQ, INTER), lambda b, c: (b, c, 0)),
            pl.BlockSpec((1, SQ, INTER), lambda b, c: (b, c, 0)),
            pl.BlockSpec((1, SQ, INTER), lambda b, c: (b, c, 0)),
            pl.BlockSpec((1, 2 * D_STATE, SQ), lambda b, c: (b, 0, c)),
            pl.BlockSpec((D_STATE, INTER), lambda b, c: (0, 0)),
            pl.BlockSpec((INTER, D_MODEL), lambda b, c: (0, 0)),
            pl.BlockSpec((1, SQ, D_MODEL), lambda b, c: (b, c, 0)),
            pl.BlockSpec((1, D_MODEL), lambda b, c: (0, 0)),
        ],
        out_specs=pl.BlockSpec((1, SQ, D_MODEL), lambda b, c: (b, c, 0)),
        out_shape=jax.ShapeDtypeStruct((B_SZ, SEQ, D_MODEL), jnp.float32),
        scratch_shapes=[
            pltpu.VMEM((SQ, INTER), jnp.float32),
            pltpu.VMEM((D_STATE, INTER), jnp.float32),
        ],
        compiler_params=pltpu.CompilerParams(
            dimension_semantics=("parallel", "arbitrary"),
            vmem_limit_bytes=48 * 1024 * 1024,
        ),
        name="mamba_scan",
        interpret=interpret,
    )(dt, xt, sg, z1, bct, atT, owT, h, nfw)


def kernel(x, in_w, conv_w, conv_b, x_w, dt_w, dt_b, A_log, D, out_w, norm_w,
           norm_f_w):
    f32 = jnp.float32
    h = x
    nfw = norm_f_w.reshape(1, D_MODEL)
    for l in range(N_LAYERS):
        inwT = in_w[l].T.astype(jnp.bfloat16)
        cwT = conv_w[l].T
        cb = conv_b[l].reshape(1, INTER)
        xwT = jnp.pad(x_w[l].T, ((0, 0), (0, 128 - (DT_RANK + 2 * D_STATE)))
                      ).astype(jnp.bfloat16)
        dtwT = dt_w[l].T.astype(f32)
        dtb = dt_b[l].reshape(1, INTER)
        dvec = D[l].reshape(1, INTER)
        nw = norm_w[l].reshape(1, D_MODEL)
        atT = (-1.4426950408889634 * jnp.exp(A_log[l])).T.astype(f32)
        owT = out_w[l].T.astype(jnp.bfloat16)

        dt_a, xt_a, sg_a, z1_a, bct = _layer_parallel(
            h, nw, inwT, cwT, cb, xwT, dtwT, dtb, dvec)
        h = _layer_scan(h, dt_a, xt_a, sg_a, z1_a, bct, atT, owT, nfw,
                        final=(l == N_LAYERS - 1))
    return h
